# baseline (speedup 1.0000x reference)
import jax
import jax.numpy as jnp
from jax.experimental import pallas as pl
from jax.experimental.pallas import tpu as pltpu

_D = 4096
_NR = 1024
_R_OUT = 64
_KT = 1024
_BM = 256
_NK = _D // _KT
_HB = 4


def _fused_kernel(wids_ref, x_ref, w_ref, out_ref, acc_ref):
    h = pl.program_id(0)
    k = pl.program_id(1)
    i = pl.program_id(2)

    xb = x_ref[...].astype(jnp.bfloat16)
    part = jnp.dot(xb, w_ref[0], preferred_element_type=jnp.float32)

    @pl.when(k == 0)
    def _init():
        acc_ref[i] = part

    @pl.when(k > 0)
    def _accum():
        acc_ref[i] += part

    @pl.when(k == _NK - 1)
    def _finish():
        shift = jnp.where(h == 0, 6, 4)
        rmask = jnp.where(h == 0, 63, 15)
        acc = acc_ref[i]
        wid = wids_ref[0, 0, :]
        lane_e = jax.lax.broadcasted_iota(jnp.int32, (_BM, _NR), 1) >> shift
        masked = jnp.where(wid[:, None] == lane_e, acc, 0.0).astype(jnp.bfloat16)
        gi = jax.lax.broadcasted_iota(jnp.int32, (_NR, _R_OUT), 0)
        gj = jax.lax.broadcasted_iota(jnp.int32, (_NR, _R_OUT), 1)
        fold = ((gi & rmask) == gj).astype(jnp.bfloat16)
        out_ref[...] = jnp.dot(masked, fold, preferred_element_type=jnp.float32)


def kernel(x, wids_large, wids_small, lora_A_large, lora_A_small):
    b_l = wids_large.shape[0]
    b_s = wids_small.shape[0]
    n_l, d, r_l = lora_A_large.shape
    n_s, _, r_s = lora_A_small.shape
    nblk = (b_l + b_s) // _BM

    wids3 = jnp.concatenate([wids_large, wids_small]).reshape(nblk, 1, _BM)
    wl = lora_A_large.transpose(1, 0, 2).reshape(d, n_l * r_l).astype(jnp.bfloat16)
    ws = lora_A_small.transpose(1, 0, 2).reshape(d, n_s * r_s).astype(jnp.bfloat16)
    w = jnp.stack([wl, ws])

    out = pl.pallas_call(
        _fused_kernel,
        grid=(2, _NK, _HB),
        in_specs=[
            pl.BlockSpec((1, 1, _BM), lambda h, k, i: (h * _HB + i, 0, 0)),
            pl.BlockSpec((_BM, _KT), lambda h, k, i: (h * _HB + i, k)),
            pl.BlockSpec((1, _KT, _NR), lambda h, k, i: (h, k, 0)),
        ],
        out_specs=pl.BlockSpec((_BM, _R_OUT), lambda h, k, i: (h * _HB + i, 0)),
        out_shape=jax.ShapeDtypeStruct((b_l + b_s, _R_OUT), jnp.float32),
        scratch_shapes=[
            pltpu.VMEM((_HB, _BM, _NR), jnp.float32),
        ],
    )(wids3, x.reshape(b_l + b_s, d), w)

    yl = out[:b_l, :r_l]
    ys = out[b_l:, :r_s]
    return (yl[:, None, :], ys[:, None, :])
